# Initial kernel scaffold; baseline (speedup 1.0000x reference)
import jax
import jax.numpy as jnp
from jax.experimental import pallas as pl


def kernel(token_ids, weight, eps):
    raise NotImplementedError("write your pallas kernel here")


# speedup vs baseline: 1.7388x; 1.7388x over previous
import functools

import jax
import jax.numpy as jnp
from jax import lax
from jax.experimental import pallas as pl
from jax.experimental.pallas import tpu as pltpu
from jax.experimental.pallas import tpu_sc as plsc

NC = 2
NS = 16
NW = NC * NS
L = 16


def _rsqrt16(x):
    i = lax.bitcast_convert_type(x, jnp.int32)
    i = jnp.int32(0x5F3759DF) - (i >> 1)
    y = lax.bitcast_convert_type(i, jnp.float32)
    for _ in range(3):
        y = y * (1.5 - 0.5 * x * y * y)
    return y


def _make_body(B, V, D, chunk):
    nvec = D // L
    b_per_w = B // NW
    nchunk = b_per_w // chunk
    mesh = plsc.VectorSubcoreMesh(core_axis_name="c", subcore_axis_name="s")

    @functools.partial(
        pl.kernel,
        mesh=mesh,
        out_type=jax.ShapeDtypeStruct((B, D), jnp.float32),
        scratch_types=[
            pltpu.VMEM((chunk,), jnp.int32),
            pltpu.VMEM((chunk, D), jnp.float32),
            pltpu.VMEM((L,), jnp.float32),
            pltpu.SemaphoreType.DMA,
        ],
    )
    def body(tok_hbm, w_hbm, eps_hbm, out_hbm, idx_v, rows_v, eps_v, sem):
        wid = lax.axis_index("s") * NC + lax.axis_index("c")
        base = wid * b_per_w
        pltpu.sync_copy(eps_hbm, eps_v)
        eps_vec = eps_v[...]
        lanes = jnp.arange(L, dtype=jnp.int32)
        perms = [lanes ^ sh for sh in (1, 2, 4, 8)]

        def row_body(r, carry):
            xs = [rows_v[r, pl.ds(L * k, L)] for k in range(nvec)]
            acc = xs[0] * xs[0]
            for k in range(1, nvec):
                acc = acc + xs[k] * xs[k]
            ss = acc
            for p in perms:
                ss = ss + ss.at[p].get(mode="promise_in_bounds")
            norm = ss * _rsqrt16(ss)
            inv = 1.0 / jnp.maximum(norm, eps_vec)
            for k in range(nvec):
                rows_v[r, pl.ds(L * k, L)] = xs[k] * inv
            return carry

        for c in range(nchunk):
            start = base + c * chunk
            pltpu.sync_copy(tok_hbm.at[pl.ds(start, chunk)], idx_v)
            pltpu.async_copy(w_hbm.at[idx_v], rows_v, sem).wait()
            lax.fori_loop(0, chunk, row_body, 0)
            pltpu.sync_copy(rows_v, out_hbm.at[pl.ds(start, chunk)])

    return body


def kernel(token_ids, weight, eps):
    Bt, H = token_ids.shape
    V, D = weight.shape
    B = Bt * H
    tok = token_ids.reshape(B).astype(jnp.int32)
    eps_arr = jnp.full((L,), eps, jnp.float32)
    body = _make_body(B, V, D, chunk=640)
    out = body(tok, weight, eps_arr)
    return out.reshape(Bt, H, D)


# speedup vs baseline: 3.1681x; 1.8220x over previous
import functools

import jax
import jax.numpy as jnp
from jax import lax
from jax.experimental import pallas as pl
from jax.experimental.pallas import tpu as pltpu
from jax.experimental.pallas import tpu_sc as plsc

NC = 2
NS = 16
NW = NC * NS
L = 16


def _rsqrt16(x):
    i = lax.bitcast_convert_type(x, jnp.int32)
    i = jnp.int32(0x5F3759DF) - (i >> 1)
    y = lax.bitcast_convert_type(i, jnp.float32)
    for _ in range(3):
        y = y * (1.5 - 0.5 * x * y * y)
    return y


def _make_body(B, V, D, chunk, nbuf=3, unroll=4):
    nvec = D // L
    b_per_w = B // NW
    nchunk = b_per_w // chunk
    mesh = plsc.VectorSubcoreMesh(core_axis_name="c", subcore_axis_name="s")

    @functools.partial(
        pl.kernel,
        mesh=mesh,
        out_type=jax.ShapeDtypeStruct((B, D), jnp.float32),
        scratch_types=[pltpu.VMEM((chunk,), jnp.int32)] * nbuf
        + [pltpu.VMEM((chunk, D), jnp.float32)] * nbuf
        + [pltpu.VMEM((L,), jnp.float32)]
        + [pltpu.SemaphoreType.DMA] * (2 * nbuf),
    )
    def body(tok_hbm, w_hbm, eps_hbm, out_hbm, *scratch):
        idx_v = scratch[:nbuf]
        rows_v = scratch[nbuf:2 * nbuf]
        eps_v = scratch[2 * nbuf]
        g_sem = scratch[2 * nbuf + 1:3 * nbuf + 1]
        o_sem = scratch[3 * nbuf + 1:]
        wid = lax.axis_index("s") * NC + lax.axis_index("c")
        base = wid * b_per_w
        pltpu.sync_copy(eps_hbm, eps_v)
        eps_vec = eps_v[...]
        lanes = jnp.arange(L, dtype=jnp.int32)
        perms = [lanes ^ sh for sh in (1, 2, 4, 8)]

        def start_gather(c):
            b = c % nbuf
            pltpu.sync_copy(tok_hbm.at[pl.ds(base + c * chunk, chunk)],
                            idx_v[b])
            return pltpu.async_copy(w_hbm.at[idx_v[b]], rows_v[b], g_sem[b])

        def make_row_body(b):
            def row_body(r, carry):
                xs = [rows_v[b][r, pl.ds(L * k, L)] for k in range(nvec)]
                acc = xs[0] * xs[0]
                for k in range(1, nvec):
                    acc = acc + xs[k] * xs[k]
                ss = acc
                for p in perms:
                    ss = ss + ss.at[p].get(mode="promise_in_bounds")
                norm = ss * _rsqrt16(ss)
                inv = 1.0 / jnp.maximum(norm, eps_vec)
                for k in range(nvec):
                    rows_v[b][r, pl.ds(L * k, L)] = xs[k] * inv
                return carry

            return row_body

        gathers = {}
        outs = {}
        for c in range(min(nbuf - 1, nchunk)):
            gathers[c] = start_gather(c)
        for c in range(nchunk):
            b = c % nbuf
            gathers.pop(c).wait()
            lax.fori_loop(0, chunk, make_row_body(b), 0, unroll=unroll)
            outs[c] = pltpu.async_copy(
                rows_v[b], out_hbm.at[pl.ds(base + c * chunk, chunk)],
                o_sem[b])
            cn = c + nbuf - 1
            if cn < nchunk:
                bn = cn % nbuf
                if cn - nbuf >= 0:
                    outs.pop(cn - nbuf).wait()
                gathers[cn] = start_gather(cn)
        for c in sorted(outs):
            outs.pop(c).wait()

    return body


def kernel(token_ids, weight, eps):
    Bt, H = token_ids.shape
    V, D = weight.shape
    B = Bt * H
    tok = token_ids.reshape(B).astype(jnp.int32)
    eps_arr = jnp.full((L,), eps, jnp.float32)
    body = _make_body(B, V, D, chunk=256)
    out = body(tok, weight, eps_arr)
    return out.reshape(Bt, H, D)
